# Initial kernel scaffold; baseline (speedup 1.0000x reference)
import jax
import jax.numpy as jnp
from jax.experimental import pallas as pl


def kernel(reference, attr, weights, batch_index):
    raise NotImplementedError("write your pallas kernel here")


# speedup vs baseline: 2.2776x; 2.2776x over previous
import jax
import jax.numpy as jnp
from jax import lax
from jax.experimental import pallas as pl
from jax.experimental.pallas import tpu as pltpu
from jax.experimental.pallas import tpu_sc as plsc

N = 100000
F = 128
B = 256
NC = 2
NS = 16
L = 16

CH = 400
G = N // CH
NW = NC * NS
ROWS_OUT = B // NS


def _sc_body(attr_hbm, w_hbm, bi_hbm, out_hbm, chunk_v, wv_v, biv_v,
             acc, red, tmp, shared):
    c = lax.axis_index("c")
    s = lax.axis_index("s")
    t = s * NC + c

    zero = jnp.zeros((L,), jnp.float32)

    def zrow(r, carry):
        for f in range(F // L):
            acc[r, pl.ds(f * L, L)] = zero
        return carry

    lax.fori_loop(0, B, zrow, 0)

    n_my = (G - t + NW - 1) // NW

    def chunk_body(i, carry):
        g = t + i * NW
        r0 = g * CH
        pltpu.sync_copy(attr_hbm.at[pl.ds(r0, CH)], chunk_v)
        pltpu.sync_copy(w_hbm.at[pl.ds(r0, CH)], wv_v)
        pltpu.sync_copy(bi_hbm.at[pl.ds(r0, CH)], biv_v)

        def row16(r16, rcarry):
            rr = r16 * L
            bv = biv_v[pl.ds(rr, L)]
            wv16 = wv_v[pl.ds(rr, L)]
            for l in range(L):
                b = bv[l]
                wr = wv16[l]
                for f in range(F // L):
                    plsc.addupdate(acc.at[b, pl.ds(f * L, L)],
                                   chunk_v[rr + l, pl.ds(f * L, L)] * wr)
            return rcarry

        lax.fori_loop(0, CH // L, row16, 0)
        return carry

    lax.fori_loop(0, n_my, chunk_body, 0)

    pltpu.sync_copy(acc, shared.at[s])
    plsc.subcore_barrier()

    ro = s * ROWS_OUT
    pltpu.sync_copy(shared.at[0, pl.ds(ro, ROWS_OUT)], red)

    def redj(j, carry):
        pltpu.sync_copy(shared.at[j, pl.ds(ro, ROWS_OUT)], tmp)
        for r in range(ROWS_OUT):
            for f in range(F // L):
                sl = pl.ds(f * L, L)
                red[r, sl] = red[r, sl] + tmp[r, sl]
        return carry

    lax.fori_loop(1, NS, redj, 0)
    pltpu.sync_copy(red, out_hbm.at[c, pl.ds(ro, ROWS_OUT)])


def _tc_add(parts_ref, out_ref):
    out_ref[...] = parts_ref[0] + parts_ref[1]


@jax.jit
def _pool(attr, w, bi):
    mesh = plsc.VectorSubcoreMesh(core_axis_name="c", subcore_axis_name="s",
                                  num_cores=NC, num_subcores=NS)
    parts = pl.kernel(
        _sc_body,
        out_type=jax.ShapeDtypeStruct((NC, B, F), jnp.float32),
        mesh=mesh,
        scratch_types=[
            pltpu.VMEM((CH, F), jnp.float32),
            pltpu.VMEM((CH,), jnp.float32),
            pltpu.VMEM((CH,), jnp.int32),
            pltpu.VMEM((B, F), jnp.float32),
            pltpu.VMEM((ROWS_OUT, F), jnp.float32),
            pltpu.VMEM((ROWS_OUT, F), jnp.float32),
            pltpu.VMEM_SHARED((NS, B, F), jnp.float32),
        ],
    )(attr, w, bi)
    return pl.pallas_call(
        _tc_add,
        out_shape=jax.ShapeDtypeStruct((B, F), jnp.float32),
    )(parts)


def kernel(reference, attr, weights, batch_index):
    del reference
    return _pool(attr, weights.reshape(-1).astype(jnp.float32),
                 batch_index.astype(jnp.int32))


# speedup vs baseline: 4.2569x; 1.8690x over previous
import jax
import jax.numpy as jnp
from jax import lax
from jax.experimental import pallas as pl
from jax.experimental.pallas import tpu as pltpu
from jax.experimental.pallas import tpu_sc as plsc

N = 100000
F = 128
B = 256
NC = 2
NS = 16
L = 16

CH = 400
G = N // CH
NW = NC * NS
ROWS_OUT = B // NS


def _sc_body(attr_hbm, w_hbm, bi_hbm, out_hbm, chunk_v, wv_v, biv_v, run_v,
             acc, red, tmp, shared):
    c = lax.axis_index("c")
    s = lax.axis_index("s")
    t = s * NC + c

    zero = jnp.zeros((L,), jnp.float32)

    def zrow(r, carry):
        for f in range(F // L):
            acc[r, pl.ds(f * L, L)] = zero
        return carry

    lax.fori_loop(0, B, zrow, 0)

    n_my = (G - t + NW - 1) // NW

    def chunk_body(i, carry):
        g = t + i * NW
        r0 = g * CH
        pltpu.sync_copy(attr_hbm.at[pl.ds(r0, CH)], chunk_v)
        pltpu.sync_copy(w_hbm.at[pl.ds(r0, CH)], wv_v)
        pltpu.sync_copy(bi_hbm.at[pl.ds(r0, CH)], biv_v)

        nf = F // L
        zvec = jnp.zeros((L,), jnp.float32)

        for f in range(nf):
            run_v[pl.ds(f * L, L)] = zvec

        def flush_if(pred, bp, accs):
            @pl.when(pred)
            def _():
                for f in range(nf):
                    plsc.addupdate(acc.at[bp, pl.ds(f * L, L)], accs[f])
            keep = jnp.where(pred, 0.0, 1.0).astype(jnp.float32)
            return [a * keep for a in accs]

        def group(r16, bp):
            rr = r16 * L
            bv = biv_v[pl.ds(rr, L)]
            wv16 = wv_v[pl.ds(rr, L)]
            accs = [run_v[pl.ds(f * L, L)] for f in range(nf)]
            for l in range(L):
                b = bv[l]
                wr = wv16[l]
                accs = flush_if(b != bp, bp, accs)
                bp = b
                for f in range(nf):
                    accs[f] = accs[f] + chunk_v[rr + l, pl.ds(f * L, L)] * wr
            for f in range(nf):
                run_v[pl.ds(f * L, L)] = accs[f]
            return bp

        bp0 = biv_v[pl.ds(0, L)][0]
        bp_end = lax.fori_loop(0, CH // L, group, bp0)
        for f in range(nf):
            plsc.addupdate(acc.at[bp_end, pl.ds(f * L, L)],
                           run_v[pl.ds(f * L, L)])
        return carry

    lax.fori_loop(0, n_my, chunk_body, 0)

    pltpu.sync_copy(acc, shared.at[s])
    plsc.subcore_barrier()

    ro = s * ROWS_OUT
    pltpu.sync_copy(shared.at[0, pl.ds(ro, ROWS_OUT)], red)

    def redj(j, carry):
        pltpu.sync_copy(shared.at[j, pl.ds(ro, ROWS_OUT)], tmp)
        for r in range(ROWS_OUT):
            for f in range(F // L):
                sl = pl.ds(f * L, L)
                red[r, sl] = red[r, sl] + tmp[r, sl]
        return carry

    lax.fori_loop(1, NS, redj, 0)
    pltpu.sync_copy(red, out_hbm.at[c, pl.ds(ro, ROWS_OUT)])


def _tc_add(parts_ref, out_ref):
    out_ref[...] = parts_ref[0] + parts_ref[1]


@jax.jit
def _pool(attr, w, bi):
    mesh = plsc.VectorSubcoreMesh(core_axis_name="c", subcore_axis_name="s",
                                  num_cores=NC, num_subcores=NS)
    parts = pl.kernel(
        _sc_body,
        out_type=jax.ShapeDtypeStruct((NC, B, F), jnp.float32),
        mesh=mesh,
        scratch_types=[
            pltpu.VMEM((CH, F), jnp.float32),
            pltpu.VMEM((CH,), jnp.float32),
            pltpu.VMEM((CH,), jnp.int32),
            pltpu.VMEM((F,), jnp.float32),
            pltpu.VMEM((B, F), jnp.float32),
            pltpu.VMEM((ROWS_OUT, F), jnp.float32),
            pltpu.VMEM((ROWS_OUT, F), jnp.float32),
            pltpu.VMEM_SHARED((NS, B, F), jnp.float32),
        ],
    )(attr, w, bi)
    return pl.pallas_call(
        _tc_add,
        out_shape=jax.ShapeDtypeStruct((B, F), jnp.float32),
    )(parts)


def kernel(reference, attr, weights, batch_index):
    del reference
    return _pool(attr, weights.reshape(-1).astype(jnp.float32),
                 batch_index.astype(jnp.int32))


# speedup vs baseline: 4.8067x; 1.1291x over previous
import jax
import jax.numpy as jnp
from jax import lax
from jax.experimental import pallas as pl
from jax.experimental.pallas import tpu as pltpu
from jax.experimental.pallas import tpu_sc as plsc

N = 100000
F = 128
B = 256
NC = 2
NS = 16
L = 16

CH = 400
G = N // CH
HB = B // NC
ROWS_OUT = HB // NS
NFV = F // L


def _sc_body(attr_hbm, w_hbm, bi_hbm, out_hbm, chunk_v, wv_v, biv_v, run_v,
             acc, red, tmp, shared):
    c = lax.axis_index("c")
    s = lax.axis_index("s")
    lo = c * HB

    zvec = jnp.zeros((L,), jnp.float32)

    def zrow(r, carry):
        for f in range(NFV):
            acc[lo + r, pl.ds(f * L, L)] = zvec
        return carry

    lax.fori_loop(0, HB, zrow, 0)

    n_my = (G - s + NS - 1) // NS

    def chunk_body(i, carry):
        g = s + i * NS
        r0 = g * CH
        pltpu.sync_copy(bi_hbm.at[pl.ds(r0, CH)], biv_v)
        b_lo = biv_v[pl.ds(0, L)][0]
        b_hi = biv_v[pl.ds(CH - L, L)][L - 1]
        process = jnp.logical_and(b_hi >= lo, b_lo < lo + HB)

        @pl.when(process)
        def _():
            pltpu.sync_copy(attr_hbm.at[pl.ds(r0, CH)], chunk_v)
            pltpu.sync_copy(w_hbm.at[pl.ds(r0, CH)], wv_v)
            for f in range(NFV):
                run_v[pl.ds(f * L, L)] = zvec

            def group(r16, bp):
                rr = r16 * L
                bv = biv_v[pl.ds(rr, L)]
                wv16 = wv_v[pl.ds(rr, L)]
                b_first = bv[0]
                b_last = bv[L - 1]

                @pl.when(b_first == b_last)
                def fast():
                    @pl.when(b_first != bp)
                    def flush():
                        for f in range(NFV):
                            sl = pl.ds(f * L, L)
                            plsc.addupdate(acc.at[bp, sl], run_v[sl])
                            run_v[sl] = zvec

                    accs = [run_v[pl.ds(f * L, L)] for f in range(NFV)]
                    for l in range(L):
                        wr = wv16[l]
                        for f in range(NFV):
                            accs[f] = accs[f] + chunk_v[rr + l,
                                                        pl.ds(f * L, L)] * wr
                    for f in range(NFV):
                        run_v[pl.ds(f * L, L)] = accs[f]

                @pl.when(b_first != b_last)
                def slow():
                    bpl = bp
                    accs = [run_v[pl.ds(f * L, L)] for f in range(NFV)]
                    for l in range(L):
                        b = bv[l]
                        wr = wv16[l]
                        pred = b != bpl

                        @pl.when(pred)
                        def flush(bpl=bpl, accs=list(accs)):
                            for f in range(NFV):
                                plsc.addupdate(acc.at[bpl, pl.ds(f * L, L)],
                                               accs[f])

                        keep = jnp.where(pred, 0.0, 1.0).astype(jnp.float32)
                        for f in range(NFV):
                            accs[f] = (accs[f] * keep
                                       + chunk_v[rr + l, pl.ds(f * L, L)] * wr)
                        bpl = b
                    for f in range(NFV):
                        run_v[pl.ds(f * L, L)] = accs[f]

                return b_last

            bp_end = lax.fori_loop(0, CH // L, group, b_lo)
            for f in range(NFV):
                sl = pl.ds(f * L, L)
                plsc.addupdate(acc.at[bp_end, sl], run_v[sl])

        return carry

    lax.fori_loop(0, n_my, chunk_body, 0)

    pltpu.sync_copy(acc.at[pl.ds(lo, HB)], shared.at[s])
    plsc.subcore_barrier()

    ro = s * ROWS_OUT
    pltpu.sync_copy(shared.at[0, pl.ds(ro, ROWS_OUT)], red)

    def redj(j, carry):
        pltpu.sync_copy(shared.at[j, pl.ds(ro, ROWS_OUT)], tmp)
        for r in range(ROWS_OUT):
            for f in range(NFV):
                sl = pl.ds(f * L, L)
                red[r, sl] = red[r, sl] + tmp[r, sl]
        return carry

    lax.fori_loop(1, NS, redj, 0)
    pltpu.sync_copy(red, out_hbm.at[pl.ds(lo + ro, ROWS_OUT)])


@jax.jit
def _pool(attr, w, bi):
    mesh = plsc.VectorSubcoreMesh(core_axis_name="c", subcore_axis_name="s",
                                  num_cores=NC, num_subcores=NS)
    return pl.kernel(
        _sc_body,
        out_type=jax.ShapeDtypeStruct((B, F), jnp.float32),
        mesh=mesh,
        scratch_types=[
            pltpu.VMEM((CH, F), jnp.float32),
            pltpu.VMEM((CH,), jnp.float32),
            pltpu.VMEM((CH,), jnp.int32),
            pltpu.VMEM((F,), jnp.float32),
            pltpu.VMEM((B, F), jnp.float32),
            pltpu.VMEM((ROWS_OUT, F), jnp.float32),
            pltpu.VMEM((ROWS_OUT, F), jnp.float32),
            pltpu.VMEM_SHARED((NS, HB, F), jnp.float32),
        ],
    )(attr, w, bi)


def kernel(reference, attr, weights, batch_index):
    del reference
    return _pool(attr, weights.reshape(-1).astype(jnp.float32),
                 batch_index.astype(jnp.int32))


# speedup vs baseline: 6.8931x; 1.4341x over previous
import jax
import jax.numpy as jnp
from jax import lax
from jax.experimental import pallas as pl
from jax.experimental.pallas import tpu as pltpu
from jax.experimental.pallas import tpu_sc as plsc

N = 100000
F = 128
B = 256
NC = 2
NS = 16
L = 16

CH = 160
G = N // CH
NMAX = (G + NS - 1) // NS
HB = B // NC
ROWS_OUT = HB // NS
NFV = F // L


def _sc_body(attr_hbm, w_hbm, bi_hbm, out_hbm, chunk0, chunk1, wv0, wv1, bib,
             run_v, acc, red, tmp, shared, sem0, sem1, sem_bi):
    c = lax.axis_index("c")
    s = lax.axis_index("s")
    lo = c * HB

    zvec = jnp.zeros((L,), jnp.float32)
    n_my = (G - s + NS - 1) // NS

    def bi_start(i, carry):
        g = s + i * NS
        pltpu.async_copy(bi_hbm.at[pl.ds(g * CH, CH)],
                         bib.at[pl.ds(i * CH, CH)], sem_bi)
        return carry

    lax.fori_loop(0, n_my, bi_start, 0)

    def zrow(r, carry):
        for f in range(NFV):
            acc[lo + r, pl.ds(f * L, L)] = zvec
        return carry

    lax.fori_loop(0, HB, zrow, 0)

    def bi_drain(i, carry):
        pltpu.make_async_copy(bi_hbm.at[pl.ds(s * CH, CH)],
                              bib.at[pl.ds(i * CH, CH)], sem_bi).wait()
        return carry

    lax.fori_loop(0, n_my, bi_drain, 0)

    bufs = ((chunk0, wv0, sem0), (chunk1, wv1, sem1))

    def flags(i):
        off = i * CH
        b_lo = bib[pl.ds(off, L)][0]
        b_hi = bib[pl.ds(off + CH - L, L)][L - 1]
        return b_lo, jnp.logical_and(b_hi >= lo, b_lo < lo + HB)

    def start_dma(i, buf):
        chunk_v, wv_v, sem = buf
        r0 = (s + i * NS) * CH
        pltpu.async_copy(attr_hbm.at[pl.ds(r0, CH)], chunk_v, sem)
        pltpu.async_copy(w_hbm.at[pl.ds(r0, CH)], wv_v, sem)

    def maybe_start(i, buf):
        _, proc = flags(i)

        @pl.when(jnp.logical_and(i < n_my, proc))
        def _():
            start_dma(i, buf)

    def process_chunk(i, buf):
        chunk_v, wv_v, sem = buf
        b_lo, proc = flags(i)

        @pl.when(jnp.logical_and(i < n_my, proc))
        def _():
            r0 = (s + i * NS) * CH
            pltpu.make_async_copy(attr_hbm.at[pl.ds(r0, CH)], chunk_v,
                                  sem).wait()
            pltpu.make_async_copy(w_hbm.at[pl.ds(r0, CH)], wv_v, sem).wait()
            for f in range(NFV):
                run_v[pl.ds(f * L, L)] = zvec
            off = i * CH

            def group(r16, bp):
                rr = r16 * L
                bv = bib[pl.ds(off + rr, L)]
                wv16 = wv_v[pl.ds(rr, L)]
                b_first = bv[0]
                b_last = bv[L - 1]

                @pl.when(b_first == b_last)
                def fast():
                    @pl.when(b_first != bp)
                    def flush():
                        for f in range(NFV):
                            sl = pl.ds(f * L, L)
                            plsc.addupdate(acc.at[bp, sl], run_v[sl])
                            run_v[sl] = zvec

                    accs = [run_v[pl.ds(f * L, L)] for f in range(NFV)]
                    for l in range(L):
                        wr = wv16[l]
                        for f in range(NFV):
                            accs[f] = accs[f] + chunk_v[rr + l,
                                                        pl.ds(f * L, L)] * wr
                    for f in range(NFV):
                        run_v[pl.ds(f * L, L)] = accs[f]

                @pl.when(b_first != b_last)
                def slow():
                    bpl = bp
                    accs = [run_v[pl.ds(f * L, L)] for f in range(NFV)]
                    for l in range(L):
                        b = bv[l]
                        wr = wv16[l]
                        pred = b != bpl

                        @pl.when(pred)
                        def flush(bpl=bpl, accs=list(accs)):
                            for f in range(NFV):
                                plsc.addupdate(acc.at[bpl, pl.ds(f * L, L)],
                                               accs[f])

                        keep = jnp.where(pred, 0.0, 1.0).astype(jnp.float32)
                        for f in range(NFV):
                            accs[f] = (accs[f] * keep
                                       + chunk_v[rr + l, pl.ds(f * L, L)] * wr)
                        bpl = b
                    for f in range(NFV):
                        run_v[pl.ds(f * L, L)] = accs[f]

                return b_last

            bp_end = lax.fori_loop(0, CH // L, group, b_lo)
            for f in range(NFV):
                sl = pl.ds(f * L, L)
                plsc.addupdate(acc.at[bp_end, sl], run_v[sl])

    maybe_start(0, bufs[0])

    def outer(o, carry):
        i0 = o * 2
        maybe_start(i0 + 1, bufs[1])
        process_chunk(i0, bufs[0])
        maybe_start(i0 + 2, bufs[0])
        process_chunk(i0 + 1, bufs[1])
        return carry

    lax.fori_loop(0, NMAX // 2, outer, 0)

    pltpu.sync_copy(acc.at[pl.ds(lo, HB)], shared.at[s])
    plsc.subcore_barrier()

    ro = s * ROWS_OUT
    pltpu.sync_copy(shared.at[0, pl.ds(ro, ROWS_OUT)], red)

    def redj(j, carry):
        pltpu.sync_copy(shared.at[j, pl.ds(ro, ROWS_OUT)], tmp)
        for r in range(ROWS_OUT):
            for f in range(NFV):
                sl = pl.ds(f * L, L)
                red[r, sl] = red[r, sl] + tmp[r, sl]
        return carry

    lax.fori_loop(1, NS, redj, 0)
    pltpu.sync_copy(red, out_hbm.at[pl.ds(lo + ro, ROWS_OUT)])


@jax.jit
def _pool(attr, w, bi):
    mesh = plsc.VectorSubcoreMesh(core_axis_name="c", subcore_axis_name="s",
                                  num_cores=NC, num_subcores=NS)
    return pl.kernel(
        _sc_body,
        out_type=jax.ShapeDtypeStruct((B, F), jnp.float32),
        mesh=mesh,
        scratch_types=[
            pltpu.VMEM((CH, F), jnp.float32),
            pltpu.VMEM((CH, F), jnp.float32),
            pltpu.VMEM((CH,), jnp.float32),
            pltpu.VMEM((CH,), jnp.float32),
            pltpu.VMEM((NMAX * CH,), jnp.int32),
            pltpu.VMEM((F,), jnp.float32),
            pltpu.VMEM((B, F), jnp.float32),
            pltpu.VMEM((ROWS_OUT, F), jnp.float32),
            pltpu.VMEM((ROWS_OUT, F), jnp.float32),
            pltpu.VMEM_SHARED((NS, HB, F), jnp.float32),
            pltpu.SemaphoreType.DMA,
            pltpu.SemaphoreType.DMA,
            pltpu.SemaphoreType.DMA,
        ],
    )(attr, w, bi)


def kernel(reference, attr, weights, batch_index):
    del reference
    return _pool(attr, weights.reshape(-1).astype(jnp.float32),
                 batch_index.astype(jnp.int32))
